# baseline (speedup 1.0000x reference)
import jax
import jax.numpy as jnp
from jax import lax
from jax.experimental import pallas as pl
from jax.experimental.pallas import tpu as pltpu
from jax.experimental.pallas import tpu_sc as plsc

N_SEL = 40
NC = 1000
NROWS = 16384
NW = 32
ROWS_PER_W = NROWS // NW
BATCH = 16
NB = ROWS_PER_W // BATCH
G = 63
NCH = 63
TSTRIDE = 65
BIG = 4096
R_ILV = 4
SLOT = BATCH * NC + 16
OSLOT = BATCH * N_SEL


def _sc_body(lik_hbm, z_hbm, out_hbm,
             likbuf, zbuf, outstage, sem_in, sem_out, *bufs):
    tbufs = bufs[0:R_ILV]
    wvs = bufs[R_ILV:2 * R_ILV]
    wis = bufs[2 * R_ILV:3 * R_ILV]
    w = lax.axis_index("s") * 2 + lax.axis_index("c")
    iota = lax.broadcasted_iota(jnp.int32, (16,), 0)
    base65 = iota * TSTRIDE
    i63 = iota * G
    neg1 = jnp.full((16,), -1.0, jnp.float32)
    zeros_f = jnp.zeros((16,), jnp.float32)
    zeros_i = jnp.zeros((16,), jnp.int32)
    lane0 = iota == 0
    tmask = iota < 8

    for tb in tbufs:
        for off in range(0, 16 * TSTRIDE, 16):
            tb[pl.ds(off, 16)] = neg1
    for wv, wi in zip(wvs, wis):
        wv[pl.ds(32, 16)] = zeros_f
        wi[pl.ds(32, 16)] = zeros_i

    row0 = w * ROWS_PER_W

    def issue_in(bb):
        s = bb % 2
        r0 = row0 + bb * BATCH
        pltpu.async_copy(lik_hbm.at[pl.ds(r0 * NC, BATCH * NC)],
                         likbuf.at[pl.ds(s * SLOT, BATCH * NC)], sem_in)
        pltpu.async_copy(z_hbm.at[pl.ds(r0 * NC, BATCH * NC)],
                         zbuf.at[pl.ds(s * SLOT, BATCH * NC)], sem_in)

    def drain_in():
        pltpu.make_async_copy(lik_hbm.at[pl.ds(0, BATCH * NC)],
                              likbuf.at[pl.ds(0, BATCH * NC)], sem_in).wait()
        pltpu.make_async_copy(z_hbm.at[pl.ds(0, BATCH * NC)],
                              zbuf.at[pl.ds(0, BATCH * NC)], sem_in).wait()

    def drain_out():
        pltpu.make_async_copy(outstage.at[pl.ds(0, OSLOT)],
                              out_hbm.at[pl.ds(0, OSLOT)], sem_out).wait()

    issue_in(0)

    def batch_body(b, carry):
        drain_in()

        @pl.when(b + 1 < NB)
        def _issue_next():
            issue_in(b + 1)

        @pl.when(b >= 1)
        def _drain_prev_out():
            drain_out()

        s = b % 2
        inoff = s * SLOT
        ooff = s * OSLOT
        r0 = row0 + b * BATCH

        def group_body(p, carry2):
            offs = [(R_ILV * p + i) * NC for i in range(R_ILV)]

            def s1(c, st):
                idxv = i63 + c
                valid = idxv < NC
                out = []
                for i in range(R_ILV):
                    v = plsc.load_gather(likbuf, [idxv + (offs[i] + inoff)])
                    v = jnp.where(valid, v, -1.0)
                    out.append(jnp.maximum(st[i], v))
                    plsc.store_scatter(tbufs[i], [base65 + c], v)
                return tuple(out)

            st0 = tuple(jnp.full((16,), -1.0, jnp.float32)
                        for _ in range(R_ILV))
            ms = lax.fori_loop(0, NCH, s1, st0)

            def extract(tbuf, m, wv, wi, tsplat):
                M = jnp.max(m)
                lv = plsc.all_reduce_ffs(m == M)
                gb = lv * TSTRIDE
                g0 = plsc.load_gather(tbuf, [gb + iota])
                g1 = plsc.load_gather(tbuf, [gb + (iota + 16)])
                g2 = plsc.load_gather(tbuf, [gb + (iota + 32)])
                g3 = plsc.load_gather(tbuf, [gb + (iota + 48)])
                f0 = plsc.all_reduce_ffs(g0 == M)
                f1 = plsc.all_reduce_ffs(g1 == M)
                f2 = plsc.all_reduce_ffs(g2 == M)
                f3 = plsc.all_reduce_ffs(g3 == M)
                pstar = jnp.where(f0 < 16, f0,
                                  jnp.where(f1 < 16, f1 + 16,
                                            jnp.where(f2 < 16, f2 + 32,
                                                      f3 + 48)))
                I = lv * G + pstar
                plsc.store_scatter(wv, [tsplat], jnp.full((16,), M),
                                   mask=lane0)
                plsc.store_scatter(wi, [tsplat], I, mask=lane0)
                plsc.store_scatter(tbuf, [gb + pstar], neg1, mask=lane0)
                h0 = jnp.where(iota == pstar, -1.0, g0)
                h1 = jnp.where(iota + 16 == pstar, -1.0, g1)
                h2 = jnp.where(iota + 32 == pstar, -1.0, g2)
                h3 = jnp.where(iota + 48 == pstar, -1.0, g3)
                nm = jnp.maximum(jnp.maximum(h0, h1), jnp.maximum(h2, h3))
                NM = jnp.max(nm)
                return jnp.where(iota == lv, NM, m)

            def s2(t, st):
                tsplat = jnp.full((16,), t, jnp.int32)
                return tuple(
                    extract(tbufs[i], st[i], wvs[i], wis[i], tsplat)
                    for i in range(R_ILV))

            lax.fori_loop(0, N_SEL, s2, ms)

            for i in range(R_ILV):
                ob = ooff + (R_ILV * p + i) * N_SEL
                for j in range(3):
                    v = wvs[i][pl.ds(16 * j, 16)]
                    ix = wis[i][pl.ds(16 * j, 16)]
                    zg = plsc.load_gather(zbuf, [ix + (offs[i] + inoff)])
                    oz = jnp.where(v == 0.0, 0.0, zg)
                    if j < 2:
                        plsc.store_scatter(outstage,
                                           [iota + (ob + 16 * j)], oz)
                    else:
                        plsc.store_scatter(outstage, [iota + (ob + 32)],
                                           oz, mask=tmask)
            return carry2

        lax.fori_loop(0, BATCH // R_ILV, group_body, 0)
        pltpu.async_copy(outstage.at[pl.ds(ooff, OSLOT)],
                         out_hbm.at[pl.ds(r0 * N_SEL, OSLOT)], sem_out)
        return carry

    lax.fori_loop(0, NB, batch_body, 0)
    drain_out()


def kernel(pt_likelihood, z_samples):
    sb, nr, nc = pt_likelihood.shape
    lik = pt_likelihood.reshape(sb * nr * nc)
    z = z_samples.reshape(sb * nr * nc)
    mesh = plsc.VectorSubcoreMesh(core_axis_name="c", subcore_axis_name="s")
    scratch = [
        pltpu.VMEM((2 * SLOT,), jnp.float32),
        pltpu.VMEM((2 * SLOT,), jnp.float32),
        pltpu.VMEM((2 * OSLOT,), jnp.float32),
        pltpu.SemaphoreType.DMA,
        pltpu.SemaphoreType.DMA,
    ]
    scratch += [pltpu.VMEM((16 * TSTRIDE,), jnp.float32)
                for _ in range(R_ILV)]
    scratch += [pltpu.VMEM((48,), jnp.float32) for _ in range(R_ILV)]
    scratch += [pltpu.VMEM((48,), jnp.int32) for _ in range(R_ILV)]
    out = pl.kernel(
        _sc_body,
        out_type=jax.ShapeDtypeStruct((NROWS * N_SEL,), jnp.float32),
        mesh=mesh,
        compiler_params=pltpu.CompilerParams(needs_layout_passes=False),
        scratch_types=scratch,
    )(lik, z)
    return out.reshape(sb, nr, N_SEL)


# speedup vs baseline: 1.0486x; 1.0486x over previous
import jax
import jax.numpy as jnp
from jax import lax
from jax.experimental import pallas as pl
from jax.experimental.pallas import tpu as pltpu
from jax.experimental.pallas import tpu_sc as plsc

N_SEL = 40
NC = 1000
NROWS = 16384
NW = 32
ROWS_PER_W = NROWS // NW
BATCH = 16
NB = ROWS_PER_W // BATCH
NCH_FULL = 62
TSTRIDE = 65
BIG = 4096
R_ILV = 4
SLOT = BATCH * NC + 16
OSLOT = BATCH * N_SEL


def _sc_body(lik_hbm, z_hbm, out_hbm,
             likbuf, zbuf, outstage, sem_in, sem_out, *bufs):
    tbufs = bufs[0:R_ILV]
    wvs = bufs[R_ILV:2 * R_ILV]
    wis = bufs[2 * R_ILV:3 * R_ILV]
    w = lax.axis_index("s") * 2 + lax.axis_index("c")
    iota = lax.broadcasted_iota(jnp.int32, (16,), 0)
    base65 = iota * TSTRIDE
    neg1 = jnp.full((16,), -1.0, jnp.float32)
    zeros_f = jnp.zeros((16,), jnp.float32)
    zeros_i = jnp.zeros((16,), jnp.int32)
    lane0 = iota == 0
    tmask = iota < 8

    for tb in tbufs:
        for off in range(0, 16 * TSTRIDE, 16):
            tb[pl.ds(off, 16)] = neg1
    for wv, wi in zip(wvs, wis):
        wv[pl.ds(32, 16)] = zeros_f
        wi[pl.ds(32, 16)] = zeros_i

    row0 = w * ROWS_PER_W

    def issue_in(bb):
        s = bb % 2
        r0 = row0 + bb * BATCH
        pltpu.async_copy(lik_hbm.at[pl.ds(r0 * NC, BATCH * NC)],
                         likbuf.at[pl.ds(s * SLOT, BATCH * NC)], sem_in)
        pltpu.async_copy(z_hbm.at[pl.ds(r0 * NC, BATCH * NC)],
                         zbuf.at[pl.ds(s * SLOT, BATCH * NC)], sem_in)

    def drain_in():
        pltpu.make_async_copy(lik_hbm.at[pl.ds(0, BATCH * NC)],
                              likbuf.at[pl.ds(0, BATCH * NC)], sem_in).wait()
        pltpu.make_async_copy(z_hbm.at[pl.ds(0, BATCH * NC)],
                              zbuf.at[pl.ds(0, BATCH * NC)], sem_in).wait()

    def drain_out():
        pltpu.make_async_copy(outstage.at[pl.ds(0, OSLOT)],
                              out_hbm.at[pl.ds(0, OSLOT)], sem_out).wait()

    issue_in(0)

    def batch_body(b, carry):
        drain_in()

        @pl.when(b + 1 < NB)
        def _issue_next():
            issue_in(b + 1)

        @pl.when(b >= 1)
        def _drain_prev_out():
            drain_out()

        s = b % 2
        inoff = s * SLOT
        ooff = s * OSLOT
        r0 = row0 + b * BATCH

        def group_body(p, carry2):
            offs = [(R_ILV * p + i) * NC for i in range(R_ILV)]

            def s1(ci, st):
                out = st
                for u in range(2):
                    c = 2 * ci + u
                    idxv = iota + c * 16
                    nxt = []
                    for i in range(R_ILV):
                        m, mi = out[i]
                        v = plsc.load_gather(likbuf,
                                             [idxv + (offs[i] + inoff)])
                        g = v > m
                        nxt.append((jnp.where(g, v, m),
                                    jnp.where(g, idxv, mi)))
                        plsc.store_scatter(tbufs[i], [base65 + c], v)
                    out = tuple(nxt)
                return out

            st0 = tuple((jnp.full((16,), -1.0, jnp.float32), zeros_i)
                        for _ in range(R_ILV))
            st = lax.fori_loop(0, NCH_FULL // 2, s1, st0)

            tidx = iota + NCH_FULL * 16
            st_l = []
            for i in range(R_ILV):
                m, mi = st[i]
                v = jnp.where(
                    tmask,
                    plsc.load_gather(likbuf, [tidx + (offs[i] + inoff)]),
                    -1.0)
                g = v > m
                st_l.append((jnp.where(g, v, m), jnp.where(g, tidx, mi)))
                plsc.store_scatter(tbufs[i], [base65 + NCH_FULL], v)
            st = tuple(st_l)

            def rescan(tbuf, I, m, mi):
                l = I & 15
                pos = I >> 4
                plsc.store_scatter(
                    tbuf, [jnp.full((16,), l * TSTRIDE + pos, jnp.int32)],
                    neg1, mask=lane0)
                gb = l * TSTRIDE
                v0 = plsc.load_gather(tbuf, [iota + gb])
                v1 = plsc.load_gather(tbuf, [iota + (gb + 16)])
                v2 = plsc.load_gather(tbuf, [iota + (gb + 32)])
                v3 = plsc.load_gather(tbuf, [iota + (gb + 48)])
                nm = jnp.maximum(jnp.maximum(v0, v1), jnp.maximum(v2, v3))
                NM = jnp.max(nm)
                f0 = plsc.all_reduce_ffs(v0 == NM)
                f1 = plsc.all_reduce_ffs(v1 == NM)
                f2 = plsc.all_reduce_ffs(v2 == NM)
                f3 = plsc.all_reduce_ffs(v3 == NM)
                P = jnp.where(f0 < 16, f0,
                              jnp.where(f1 < 16, f1 + 16,
                                        jnp.where(f2 < 16, f2 + 32,
                                                  f3 + 48)))
                lm = iota == l
                return jnp.where(lm, NM, m), jnp.where(lm, P * 16 + l, mi)

            def s2(t, st):
                tsplat = jnp.full((16,), t, jnp.int32)
                Ms = []
                Is = []
                for i in range(R_ILV):
                    m, mi = st[i]
                    M = jnp.max(m)
                    I = jnp.min(jnp.where(m == M, mi, BIG))
                    Ms.append(M)
                    Is.append(I)
                for i in range(R_ILV):
                    plsc.store_scatter(wvs[i], [tsplat],
                                       jnp.full((16,), Ms[i]), mask=lane0)
                    plsc.store_scatter(wis[i], [tsplat],
                                       jnp.full((16,), Is[i]), mask=lane0)
                return tuple(rescan(tbufs[i], Is[i], st[i][0], st[i][1])
                             for i in range(R_ILV))

            lax.fori_loop(0, N_SEL, s2, st)

            for i in range(R_ILV):
                ob = ooff + (R_ILV * p + i) * N_SEL
                for j in range(3):
                    v = wvs[i][pl.ds(16 * j, 16)]
                    ix = wis[i][pl.ds(16 * j, 16)]
                    zg = plsc.load_gather(zbuf, [ix + (offs[i] + inoff)])
                    oz = jnp.where(v == 0.0, 0.0, zg)
                    if j < 2:
                        plsc.store_scatter(outstage,
                                           [iota + (ob + 16 * j)], oz)
                    else:
                        plsc.store_scatter(outstage, [iota + (ob + 32)],
                                           oz, mask=tmask)
            return carry2

        lax.fori_loop(0, BATCH // R_ILV, group_body, 0)
        pltpu.async_copy(outstage.at[pl.ds(ooff, OSLOT)],
                         out_hbm.at[pl.ds(r0 * N_SEL, OSLOT)], sem_out)
        return carry

    lax.fori_loop(0, NB, batch_body, 0)
    drain_out()


def kernel(pt_likelihood, z_samples):
    sb, nr, nc = pt_likelihood.shape
    lik = pt_likelihood.reshape(sb * nr * nc)
    z = z_samples.reshape(sb * nr * nc)
    mesh = plsc.VectorSubcoreMesh(core_axis_name="c", subcore_axis_name="s")
    scratch = [
        pltpu.VMEM((2 * SLOT,), jnp.float32),
        pltpu.VMEM((2 * SLOT,), jnp.float32),
        pltpu.VMEM((2 * OSLOT,), jnp.float32),
        pltpu.SemaphoreType.DMA,
        pltpu.SemaphoreType.DMA,
    ]
    scratch += [pltpu.VMEM((16 * TSTRIDE,), jnp.float32)
                for _ in range(R_ILV)]
    scratch += [pltpu.VMEM((48,), jnp.float32) for _ in range(R_ILV)]
    scratch += [pltpu.VMEM((48,), jnp.int32) for _ in range(R_ILV)]
    out = pl.kernel(
        _sc_body,
        out_type=jax.ShapeDtypeStruct((NROWS * N_SEL,), jnp.float32),
        mesh=mesh,
        compiler_params=pltpu.CompilerParams(needs_layout_passes=False),
        scratch_types=scratch,
    )(lik, z)
    return out.reshape(sb, nr, N_SEL)
